# baseline (speedup 1.0000x reference)
import functools

import jax
import jax.numpy as jnp
from jax import lax
from jax.experimental import pallas as pl
from jax.experimental.pallas import tpu as pltpu
from jax.experimental.pallas import tpu_sc as plsc

B = 16384
N = 200
N_BINS = 22
N_DIM = 4
OUT_COLS = N_BINS * N_DIM
LANES = 16
HIST_STRIDE = 32
NUM_CORES = 2
NUM_SUBCORES = 16
NW = NUM_CORES * NUM_SUBCORES
ROWS_PER_W = B // NW
GROUP = LANES
CHUNK = 64
GROUPS_PER_CHUNK = CHUNK // GROUP
CHUNKS = ROWS_PER_W // CHUNK
OUT_VREGS = 6
LUT_SIZE = 256

def _sc_hist_body(cos_hbm, lut_hbm, emb_hbm, out_hbm, in_v, out_v, hist_v,
                  lut_v, emb_v):
    wid = lax.axis_index("s") * NUM_CORES + lax.axis_index("c")
    row0 = wid * ROWS_PER_W

    pltpu.sync_copy(lut_hbm, lut_v)
    pltpu.sync_copy(emb_hbm, emb_v)

    lanes = lax.iota(jnp.int32, LANES)
    row_off = lanes * N
    cvec = lanes * HIST_STRIDE + 10
    ones = jnp.full((LANES,), 1.0, dtype=jnp.float32)
    zeros = jnp.zeros((LANES,), dtype=jnp.float32)
    colbins = [(lanes + 16 * v) >> 2 for v in range(OUT_VREGS)]
    embs = [emb_v[pl.ds(16 * v, LANES)] for v in range(OUT_VREGS)]

    @pl.loop(0, CHUNKS)
    def _chunk_loop(c):
        rbase = row0 + c * CHUNK
        pltpu.sync_copy(cos_hbm.at[pl.ds(rbase * N, CHUNK * N)], in_v)

        @pl.loop(0, GROUPS_PER_CHUNK)
        def _group_loop(g):
            for i in range(GROUP * HIST_STRIDE // LANES):
                hist_v[pl.ds(i * LANES, LANES)] = zeros

            src0 = row_off + g * (GROUP * N)

            @pl.loop(0, N, unroll=8)
            def _val_loop(n):
                vals = plsc.load_gather(in_v, [src0 + n])
                y = vals * jnp.float32(10.0)
                t = y.astype(jnp.int32)
                tf = t.astype(jnp.float32)
                frac = tf < y
                a = t + cvec
                idx = jnp.where(frac, a + 1, a)
                plsc.addupdate_scatter(hist_v, [idx], ones)

            obase = g * (GROUP * OUT_COLS)
            for r in range(GROUP):
                hbase = r * HIST_STRIDE
                for v in range(OUT_VREGS):
                    cnt = plsc.load_gather(hist_v, [colbins[v] + hbase])
                    lg = plsc.load_gather(lut_v, [cnt.astype(jnp.int32)])
                    out_v[pl.ds(obase + r * OUT_COLS + v * LANES, LANES)] = (
                        lg * embs[v])

        pltpu.sync_copy(
            out_v.at[pl.ds(0, CHUNK * OUT_COLS)],
            out_hbm.at[pl.ds(rbase * OUT_COLS, CHUNK * OUT_COLS)])


_sc_hist_kernel = None


def _get_sc_kernel():
    global _sc_hist_kernel
    if _sc_hist_kernel is None:
        mesh = plsc.VectorSubcoreMesh(
            core_axis_name="c",
            subcore_axis_name="s",
            num_cores=NUM_CORES,
            num_subcores=NUM_SUBCORES,
        )
        _sc_hist_kernel = pl.kernel(
            _sc_hist_body,
            out_type=jax.ShapeDtypeStruct((B * OUT_COLS,), jnp.float32),
            mesh=mesh,
            scratch_types=[
                pltpu.VMEM((CHUNK * N,), jnp.float32),
                pltpu.VMEM((CHUNK * OUT_COLS + LANES,), jnp.float32),
                pltpu.VMEM((GROUP * HIST_STRIDE,), jnp.float32),
                pltpu.VMEM((LUT_SIZE,), jnp.float32),
                pltpu.VMEM((OUT_VREGS * LANES,), jnp.float32),
            ],
            compiler_params=pltpu.CompilerParams(needs_layout_passes=False),
        )
    return _sc_hist_kernel


def kernel(cosine, emb):
    lut = jnp.log(jnp.arange(LUT_SIZE, dtype=jnp.float32) + 1.0)
    embv = jnp.concatenate(
        [emb.reshape(-1),
         jnp.zeros((OUT_VREGS * LANES - OUT_COLS,), jnp.float32)])
    out = _get_sc_kernel()(cosine.reshape(-1), lut, embv)
    return out.reshape(B, OUT_COLS)


# speedup vs baseline: 1.7140x; 1.7140x over previous
import jax
import jax.numpy as jnp
from jax import lax
from jax.experimental import pallas as pl
from jax.experimental.pallas import tpu as pltpu
from jax.experimental.pallas import tpu_sc as plsc

B = 16384
N = 200
N_BINS = 22
N_DIM = 4
OUT_COLS = N_BINS * N_DIM
LANES = 16
HIST_STRIDE = 33
HIST_WORDS = LANES * HIST_STRIDE
NUM_CORES = 2
NUM_SUBCORES = 16
NW = NUM_CORES * NUM_SUBCORES
ROWS_PER_W = B // NW
GROUP = LANES
CHUNK = 64
GROUPS_PER_CHUNK = CHUNK // GROUP
CHUNKS = ROWS_PER_W // CHUNK
LUT_SIZE = 256
OUT_STAGE = CHUNK * OUT_COLS
OUT_PAD = GROUP * OUT_COLS


def _sc_hist_body(cos_hbm, lut2_hbm, out_hbm, in_v, out_v, hist_v, lut2_v):
    wid = lax.axis_index("s") * NUM_CORES + lax.axis_index("c")
    row0 = wid * ROWS_PER_W

    pltpu.sync_copy(lut2_hbm, lut2_v)

    lanes = lax.iota(jnp.int32, LANES)
    row_off = lanes * N
    cvec = lanes * HIST_STRIDE + 10
    lane33 = lanes * HIST_STRIDE
    lane88 = lanes * OUT_COLS
    ones = jnp.full((LANES,), 1.0, dtype=jnp.float32)
    zeros = jnp.zeros((LANES,), dtype=jnp.float32)

    @pl.loop(0, CHUNKS)
    def _chunk_loop(c):
        rbase = row0 + c * CHUNK
        pltpu.sync_copy(cos_hbm.at[pl.ds(rbase * N, CHUNK * N)], in_v)

        @pl.loop(0, GROUPS_PER_CHUNK)
        def _group_loop(g):
            for i in range(HIST_WORDS // LANES):
                hist_v[pl.ds(i * LANES, LANES)] = zeros

            src0 = row_off + g * (GROUP * N)

            @plsc.parallel_loop(0, N, unroll=8)
            def _val_loop(n):
                vals = plsc.load_gather(in_v, [src0 + n])
                y = vals * jnp.float32(10.0)
                t = y.astype(jnp.int32)
                tf = t.astype(jnp.float32)
                a = t + cvec
                idx = jnp.where(tf < y, a + 1, a)
                plsc.addupdate_scatter(hist_v, [idx], ones)

            obase = g * (GROUP * OUT_COLS)
            outb = lane88 + obase
            for j in range(N_BINS):
                cnt = plsc.load_gather(hist_v, [lane33 + j])
                ci = cnt.astype(jnp.int32)
                for d in range(N_DIM):
                    col = 4 * j + d
                    lg = plsc.load_gather(lut2_v, [ci + col * LUT_SIZE])
                    plsc.store_scatter(out_v, [outb + col], lg)

        pltpu.sync_copy(
            out_v.at[pl.ds(0, OUT_STAGE)],
            out_hbm.at[pl.ds(rbase * OUT_COLS, OUT_STAGE)])


_sc_hist_kernel = None


def _get_sc_kernel():
    global _sc_hist_kernel
    if _sc_hist_kernel is None:
        mesh = plsc.VectorSubcoreMesh(
            core_axis_name="c",
            subcore_axis_name="s",
            num_cores=NUM_CORES,
            num_subcores=NUM_SUBCORES,
        )
        _sc_hist_kernel = pl.kernel(
            _sc_hist_body,
            out_type=jax.ShapeDtypeStruct((B * OUT_COLS,), jnp.float32),
            mesh=mesh,
            scratch_types=[
                pltpu.VMEM((CHUNK * N,), jnp.float32),
                pltpu.VMEM((OUT_STAGE + OUT_PAD,), jnp.float32),
                pltpu.VMEM((HIST_WORDS,), jnp.float32),
                pltpu.VMEM((OUT_COLS * LUT_SIZE,), jnp.float32),
            ],
            compiler_params=pltpu.CompilerParams(needs_layout_passes=False),
        )
    return _sc_hist_kernel


def kernel(cosine, emb):
    lut = jnp.log(jnp.arange(LUT_SIZE, dtype=jnp.float32) + 1.0)
    lut2 = (emb.reshape(OUT_COLS, 1) * lut.reshape(1, LUT_SIZE)).reshape(-1)
    out = _get_sc_kernel()(cosine.reshape(-1), lut2)
    return out.reshape(B, OUT_COLS)
